# baseline (speedup 1.0000x reference)
import functools

import jax
import jax.numpy as jnp
from jax import lax
from jax.experimental import pallas as pl
from jax.experimental.pallas import tpu as pltpu
from jax.experimental.pallas import tpu_sc as plsc

CHUNK = 128


def _gather_kernel(n_total, n_chunks_total, embed):
    info = plsc.get_sparse_core_info()
    nc, ns = info.num_cores, info.num_subcores
    nw = nc * ns
    n_per_w = n_total // nw
    n_ch = n_chunks_total // nw
    mesh = plsc.VectorSubcoreMesh(core_axis_name="c", subcore_axis_name="s")

    @functools.partial(
        pl.kernel,
        mesh=mesh,
        compiler_params=pltpu.CompilerParams(use_tc_tiling_on_sc=False),
        out_type=jax.ShapeDtypeStruct((n_total, embed), jnp.float32),
        scratch_types=[
            pltpu.VMEM((n_ch, CHUNK), jnp.int32),
            pltpu.VMEM((CHUNK, embed), jnp.float32),
            pltpu.SemaphoreType.DMA,
        ],
    )
    def k(idx_hbm, tab_hbm, out_hbm, idx_v, rows_v, sem):
        wid = lax.axis_index("s") * nc + lax.axis_index("c")
        row_base = wid * n_ch
        base = wid * n_per_w
        pltpu.sync_copy(idx_hbm.at[pl.ds(row_base, n_ch)], idx_v)

        def body(c, _):
            pltpu.async_copy(tab_hbm.at[idx_v.at[c]], rows_v, sem).wait()
            pltpu.sync_copy(rows_v, out_hbm.at[pl.ds(base + c * CHUNK, CHUNK)])
            return ()

        lax.fori_loop(0, n_ch, body, ())

    return k


def kernel(input_x, table):
    b, s = input_x.shape
    _, embed = table.shape
    n_total = b * s
    idx2d = input_x.reshape(n_total // CHUNK, CHUNK)
    out = _gather_kernel(n_total, n_total // CHUNK, embed)(idx2d, table)
    return out.reshape(b, s, embed)


# speedup vs baseline: 1.1104x; 1.1104x over previous
import functools

import jax
import jax.numpy as jnp
from jax import lax
from jax.experimental import pallas as pl
from jax.experimental.pallas import tpu as pltpu
from jax.experimental.pallas import tpu_sc as plsc

CHUNK = 128
NBUF = 8


def _gather_kernel(n_total, n_chunks_total, embed):
    info = plsc.get_sparse_core_info()
    nc, ns = info.num_cores, info.num_subcores
    nw = nc * ns
    n_per_w = n_total // nw
    n_ch = n_chunks_total // nw
    mesh = plsc.VectorSubcoreMesh(core_axis_name="c", subcore_axis_name="s")

    @functools.partial(
        pl.kernel,
        mesh=mesh,
        compiler_params=pltpu.CompilerParams(use_tc_tiling_on_sc=False),
        out_type=jax.ShapeDtypeStruct((n_total, embed), jnp.float32),
        scratch_types=[
            pltpu.VMEM((n_ch, CHUNK), jnp.int32),
            pltpu.VMEM((NBUF, CHUNK, embed), jnp.float32),
        ]
        + [pltpu.SemaphoreType.DMA] * (2 * NBUF),
    )
    def k(idx_hbm, tab_hbm, out_hbm, idx_v, rows_v, *sems):
        gsem, osem = sems[:NBUF], sems[NBUF:]
        wid = lax.axis_index("s") * nc + lax.axis_index("c")
        row_base = wid * n_ch
        base = wid * n_per_w
        pltpu.sync_copy(idx_hbm.at[pl.ds(row_base, n_ch)], idx_v)

        for b in range(NBUF):
            pltpu.async_copy(tab_hbm.at[idx_v.at[b]], rows_v.at[b], gsem[b])

        def body(g, _):
            c0 = g * NBUF
            for b in range(NBUF):
                c = c0 + b
                dst = out_hbm.at[pl.ds(base + c * CHUNK, CHUNK)]
                pltpu.make_async_copy(
                    tab_hbm.at[idx_v.at[c]], rows_v.at[b], gsem[b]
                ).wait()
                pltpu.async_copy(rows_v.at[b], dst, osem[b])
            for b in range(NBUF):
                c = c0 + b
                dst = out_hbm.at[pl.ds(base + c * CHUNK, CHUNK)]
                pltpu.make_async_copy(rows_v.at[b], dst, osem[b]).wait()

                @pl.when(c + NBUF < n_ch)
                def _():
                    pltpu.async_copy(
                        tab_hbm.at[idx_v.at[c + NBUF]], rows_v.at[b], gsem[b]
                    )

            return ()

        lax.fori_loop(0, n_ch // NBUF, body, ())

    return k


def kernel(input_x, table):
    b, s = input_x.shape
    _, embed = table.shape
    n_total = b * s
    idx2d = input_x.reshape(n_total // CHUNK, CHUNK)
    out = _gather_kernel(n_total, n_total // CHUNK, embed)(idx2d, table)
    return out.reshape(b, s, embed)


# speedup vs baseline: 1.1111x; 1.0006x over previous
import functools

import jax
import jax.numpy as jnp
from jax import lax
from jax.experimental import pallas as pl
from jax.experimental.pallas import tpu as pltpu
from jax.experimental.pallas import tpu_sc as plsc

CHUNK = 256
NBUF = 5


def _gather_kernel(n_total, n_chunks_total, embed):
    info = plsc.get_sparse_core_info()
    nc, ns = info.num_cores, info.num_subcores
    nw = nc * ns
    n_per_w = n_total // nw
    n_ch = n_chunks_total // nw
    mesh = plsc.VectorSubcoreMesh(core_axis_name="c", subcore_axis_name="s")

    @functools.partial(
        pl.kernel,
        mesh=mesh,
        compiler_params=pltpu.CompilerParams(use_tc_tiling_on_sc=False),
        out_type=jax.ShapeDtypeStruct((n_total, embed), jnp.float32),
        scratch_types=[
            pltpu.VMEM((n_ch, CHUNK), jnp.int32),
            pltpu.VMEM((NBUF, CHUNK, embed), jnp.float32),
        ]
        + [pltpu.SemaphoreType.DMA] * (2 * NBUF),
    )
    def k(idx_hbm, tab_hbm, out_hbm, idx_v, rows_v, *sems):
        gsem, osem = sems[:NBUF], sems[NBUF:]
        wid = lax.axis_index("s") * nc + lax.axis_index("c")
        row_base = wid * n_ch
        base = wid * n_per_w
        pltpu.sync_copy(idx_hbm.at[pl.ds(row_base, n_ch)], idx_v)

        for b in range(NBUF):
            pltpu.async_copy(tab_hbm.at[idx_v.at[b]], rows_v.at[b], gsem[b])

        def body(g, _):
            c0 = g * NBUF
            for b in range(NBUF):
                c = c0 + b
                dst = out_hbm.at[pl.ds(base + c * CHUNK, CHUNK)]
                pltpu.make_async_copy(
                    tab_hbm.at[idx_v.at[c]], rows_v.at[b], gsem[b]
                ).wait()
                pltpu.async_copy(rows_v.at[b], dst, osem[b])
            for b in range(NBUF):
                c = c0 + b
                dst = out_hbm.at[pl.ds(base + c * CHUNK, CHUNK)]
                pltpu.make_async_copy(rows_v.at[b], dst, osem[b]).wait()

                @pl.when(c + NBUF < n_ch)
                def _():
                    pltpu.async_copy(
                        tab_hbm.at[idx_v.at[c + NBUF]], rows_v.at[b], gsem[b]
                    )

            return ()

        lax.fori_loop(0, n_ch // NBUF, body, ())

    return k


def kernel(input_x, table):
    b, s = input_x.shape
    _, embed = table.shape
    n_total = b * s
    idx2d = input_x.reshape(n_total // CHUNK, CHUNK)
    out = _gather_kernel(n_total, n_total // CHUNK, embed)(idx2d, table)
    return out.reshape(b, s, embed)
